# Initial kernel scaffold; baseline (speedup 1.0000x reference)
import jax
import jax.numpy as jnp
from jax.experimental import pallas as pl


def kernel(x, edge_index, edge_weight):
    raise NotImplementedError("write your pallas kernel here")


# speedup vs baseline: 3.6556x; 3.6556x over previous
import functools

import jax
import jax.numpy as jnp
from jax import lax
from jax.experimental import pallas as pl
from jax.experimental.pallas import tpu as pltpu
from jax.experimental.pallas import tpu_sc as plsc

N_NODES = 10000
D_FEAT = 128
N_EDGES = 320000

NUM_CORES = 2
NUM_SUBCORES = 16
NUM_WORKERS = NUM_CORES * NUM_SUBCORES
CHUNK = 128
CHUNKS_PER_WORKER = 79
EDGES_PER_WORKER = CHUNKS_PER_WORKER * CHUNK
E_PAD = NUM_WORKERS * EDGES_PER_WORKER
N_PAD = 10240
ROWS_PER_TILE = N_PAD // NUM_SUBCORES
ZCHUNK = 128


@functools.partial(
    pl.kernel,
    out_type=jax.ShapeDtypeStruct((NUM_CORES * N_PAD, D_FEAT), jnp.float32),
    mesh=plsc.VectorSubcoreMesh(core_axis_name="c", subcore_axis_name="s"),
    scratch_types=[
        pltpu.VMEM_SHARED((N_PAD, D_FEAT), jnp.float32),
        pltpu.VMEM((CHUNK,), jnp.int32),
        pltpu.VMEM((CHUNK,), jnp.int32),
        pltpu.VMEM((CHUNK,), jnp.float32),
        pltpu.VMEM((CHUNK, D_FEAT), jnp.float32),
        pltpu.SemaphoreType.DMA,
    ],
)
def _conv_pass(x_hbm, src_hbm, dst_hbm, w_hbm, out_hbm,
               acc, src_v, dst_v, w_v, rows_v, sem):
    c = lax.axis_index("c")
    s = lax.axis_index("s")
    wid = c * NUM_SUBCORES + s

    zvec = jnp.zeros((16,), jnp.float32)

    def zero_rows(i, _):
        for j in range(D_FEAT // 16):
            rows_v[i, pl.ds(j * 16, 16)] = zvec
        return 0

    lax.fori_loop(0, CHUNK, zero_rows, 0)
    row0 = s * ROWS_PER_TILE
    for k in range(ROWS_PER_TILE // ZCHUNK):
        pltpu.sync_copy(rows_v.at[pl.ds(0, ZCHUNK)],
                        acc.at[pl.ds(row0 + k * ZCHUNK, ZCHUNK)])
    plsc.subcore_barrier()

    base0 = wid * EDGES_PER_WORKER

    def chunk_body(i, _):
        base = base0 + i * CHUNK
        pltpu.sync_copy(src_hbm.at[pl.ds(base, CHUNK)], src_v)
        pltpu.sync_copy(dst_hbm.at[pl.ds(base, CHUNK)], dst_v)
        pltpu.sync_copy(w_hbm.at[pl.ds(base, CHUNK)], w_v)
        pltpu.async_copy(x_hbm.at[src_v], rows_v, sem).wait()

        def scale_group(g, _):
            wvec = w_v[pl.ds(g * 16, 16)]
            for l in range(16):
                e = g * 16 + l
                wsp = wvec[l]
                for j in range(D_FEAT // 16):
                    sl = pl.ds(j * 16, 16)
                    rows_v[e, sl] = rows_v[e, sl] * wsp
            return 0

        lax.fori_loop(0, CHUNK // 16, scale_group, 0)
        pltpu.sync_copy(rows_v, acc.at[dst_v], add=True)
        return 0

    lax.fori_loop(0, CHUNKS_PER_WORKER, chunk_body, 0)
    plsc.subcore_barrier()

    out0 = c * N_PAD + row0
    for k in range(ROWS_PER_TILE // ZCHUNK):
        pltpu.sync_copy(acc.at[pl.ds(row0 + k * ZCHUNK, ZCHUNK)],
                        rows_v.at[pl.ds(0, ZCHUNK)])
        pltpu.sync_copy(rows_v.at[pl.ds(0, ZCHUNK)],
                        out_hbm.at[pl.ds(out0 + k * ZCHUNK, ZCHUNK)])


def _add_body(a_ref, b_ref, o_ref):
    o_ref[...] = a_ref[...] + b_ref[...]


def _add_relu_body(a_ref, b_ref, o_ref):
    o_ref[...] = jnp.maximum(a_ref[...] + b_ref[...], 0.0)


def _combine(p0, p1, relu):
    body = _add_relu_body if relu else _add_body
    blk = 1024
    return pl.pallas_call(
        body,
        grid=(N_PAD // blk,),
        in_specs=[pl.BlockSpec((blk, D_FEAT), lambda i: (i, 0)),
                  pl.BlockSpec((blk, D_FEAT), lambda i: (i, 0))],
        out_specs=pl.BlockSpec((blk, D_FEAT), lambda i: (i, 0)),
        out_shape=jax.ShapeDtypeStruct((N_PAD, D_FEAT), jnp.float32),
    )(p0, p1)


def kernel(x, edge_index, edge_weight):
    src = edge_index[0].astype(jnp.int32)
    dst = edge_index[1].astype(jnp.int32)
    w = edge_weight.astype(jnp.float32)
    pad = E_PAD - N_EDGES
    zpad = jnp.zeros((pad,), jnp.int32)
    src = jnp.concatenate([src, zpad])
    dst = jnp.concatenate([dst, zpad])
    w = jnp.concatenate([w, jnp.zeros((pad,), jnp.float32)])

    p = _conv_pass(x, src, dst, w)
    h = _combine(p[:N_PAD], p[N_PAD:], relu=False)
    p2 = _conv_pass(h, src, dst, w)
    out = _combine(p2[:N_PAD], p2[N_PAD:], relu=True)
    return out[:N_NODES]
